# Initial kernel scaffold; baseline (speedup 1.0000x reference)
import jax
import jax.numpy as jnp
from jax.experimental import pallas as pl


def kernel(x_F, x_S, edge_index, W_F1, b_F1, W_F2, b_F2, W_S1, b_S1, W_S2, b_S2, W_fs1, b_fs1, W_fs2, b_fs2, W_sf1, b_sf1, W_sf2, b_sf2):
    raise NotImplementedError("write your pallas kernel here")


# speedup vs baseline: 3.4213x; 3.4213x over previous
import functools

import jax
import jax.numpy as jnp
from jax import lax
from jax.experimental import pallas as pl
from jax.experimental.pallas import tpu as pltpu
from jax.experimental.pallas import tpu_sc as plsc

NC = 2
NS = 16
NW = NC * NS
L = 16
CHUNK = 128
NDUMMY = 64


def _mesh():
    return plsc.VectorSubcoreMesh(core_axis_name="c", subcore_axis_name="s")


def _chunk_loop(n_chunks, wid, nworkers, step):
    full_t = n_chunks // nworkers
    rem = n_chunks % nworkers

    def body(t, _):
        step(t * nworkers + wid)
        return 0

    lax.fori_loop(0, full_t, body, 0)
    if rem:
        @pl.when(wid < rem)
        def _():
            step(full_t * nworkers + wid)


def _block_loop(n_rows, s, fn):
    nblk = -(-n_rows // CHUNK)
    lastbase = n_rows - CHUNK

    def run(blk):
        fn(jnp.minimum(blk * CHUNK, lastbase))

    _chunk_loop(nblk, s, NS, run)


def _sp_zero(iota_hbm, zeros_v, ids_v, acc_sh, s, n_rows):
    def z(base):
        pltpu.sync_copy(iota_hbm.at[pl.ds(base, CHUNK)], ids_v)
        pltpu.sync_copy(zeros_v, acc_sh.at[ids_v])

    _block_loop(n_rows, s, z)


def _sp_writeout(iota_hbm, rows_v, ids_v, acc_sh, out_hbm, c, s, n_rows,
                 sem):
    def w(base):
        pltpu.sync_copy(iota_hbm.at[pl.ds(base, CHUNK)], ids_v)
        pltpu.async_copy(acc_sh.at[ids_v], rows_v, sem).wait()
        pltpu.sync_copy(rows_v, out_hbm.at[c, pl.ds(base, CHUNK)])

    _block_loop(n_rows, s, w)


def _sp_stage_in(iota_hbm, rows_v, ids_v, tab_hbm, tab_sh, s, n_rows):
    def w(base):
        pltpu.sync_copy(iota_hbm.at[pl.ds(base, CHUNK)], ids_v)
        pltpu.sync_copy(tab_hbm.at[pl.ds(base, CHUNK)], rows_v)
        pltpu.sync_copy(rows_v, tab_sh.at[ids_v])

    _block_loop(n_rows, s, w)


def _make_deg(n_nodes, n_edges):
    nch = n_edges // CHUNK

    @functools.partial(
        pl.kernel,
        out_type=jax.ShapeDtypeStruct((NC, n_nodes, L), jnp.float32),
        mesh=_mesh(),
        scratch_types=[
            pltpu.VMEM((CHUNK,), jnp.int32),
            pltpu.VMEM((CHUNK,), jnp.int32),
            pltpu.VMEM((CHUNK, L), jnp.float32),
            pltpu.VMEM((CHUNK, L), jnp.float32),
            pltpu.VMEM((CHUNK, L), jnp.float32),
            pltpu.VMEM_SHARED((n_nodes, L), jnp.float32),
            pltpu.SemaphoreType.DMA,
        ],
    )
    def deg_k(dst_hbm, iota_hbm, ones_hbm, zeros_hbm, out_hbm,
              idx_v, ids_v, ones_v, zeros_v, rows_v, acc_sh, sem):
        c = lax.axis_index("c")
        s = lax.axis_index("s")
        wid = s * NC + c

        pltpu.sync_copy(ones_hbm, ones_v)
        pltpu.sync_copy(zeros_hbm, zeros_v)
        _sp_zero(iota_hbm, zeros_v, ids_v, acc_sh, s, n_nodes)
        plsc.subcore_barrier()

        def step(ch):
            base = ch * CHUNK
            pltpu.sync_copy(dst_hbm.at[pl.ds(base, CHUNK)], idx_v)
            pltpu.sync_copy(ones_v, acc_sh.at[idx_v], add=True)

        _chunk_loop(nch, wid, NW, step)

        plsc.subcore_barrier()
        _sp_writeout(iota_hbm, rows_v, ids_v, acc_sh, out_hbm, c, s,
                     n_nodes, sem)

    return deg_k


def _make_prop(n_nodes, n_edges, width):
    nch = n_edges // CHUNK
    hn = n_nodes // NC
    acc_rows = hn + NDUMMY

    @functools.partial(
        pl.kernel,
        out_type=jax.ShapeDtypeStruct((NC, acc_rows, width), jnp.float32),
        mesh=_mesh(),
        scratch_types=[
            pltpu.VMEM((CHUNK,), jnp.int32),
            pltpu.VMEM((CHUNK,), jnp.int32),
            pltpu.VMEM((CHUNK,), jnp.int32),
            pltpu.VMEM((CHUNK, width), jnp.float32),
            pltpu.VMEM((CHUNK, width), jnp.float32),
            pltpu.VMEM_SHARED((acc_rows, width), jnp.float32),
            pltpu.SemaphoreType.DMA,
        ],
    )
    def prop_k(table_hbm, src_hbm, dstloc_hbm, iota_hbm, zeros_hbm,
               out_hbm, sidx_v, didx_v, ids_v, rows_v, zeros_v, acc_sh,
               sem):
        c = lax.axis_index("c")
        s = lax.axis_index("s")

        pltpu.sync_copy(zeros_hbm, zeros_v)
        _sp_zero(iota_hbm, zeros_v, ids_v, acc_sh, s, acc_rows)
        plsc.subcore_barrier()

        def step(ch):
            base = ch * CHUNK
            pltpu.sync_copy(src_hbm.at[pl.ds(base, CHUNK)], sidx_v)
            pltpu.sync_copy(dstloc_hbm.at[c, pl.ds(base, CHUNK)], didx_v)
            pltpu.async_copy(table_hbm.at[sidx_v], rows_v, sem).wait()
            pltpu.sync_copy(rows_v, acc_sh.at[didx_v], add=True)

        _chunk_loop(nch, s, NS, step)

        plsc.subcore_barrier()
        _sp_writeout(iota_hbm, rows_v, ids_v, acc_sh, out_hbm, c, s,
                     acc_rows, sem)

    return prop_k


def _make_gather4(n_nodes, n_edges, width):
    nch = n_edges // CHUNK
    out = jax.ShapeDtypeStruct((n_edges, width), jnp.float32)

    @functools.partial(
        pl.kernel,
        out_type=(out, out),
        mesh=_mesh(),
        scratch_types=[
            pltpu.VMEM((CHUNK,), jnp.int32),
            pltpu.VMEM((CHUNK, width), jnp.float32),
            pltpu.VMEM((CHUNK, width), jnp.float32),
            pltpu.SemaphoreType.DMA,
            pltpu.SemaphoreType.DMA,
        ],
    )
    def gat_k(tA, tB, src_hbm, oA, oB, idx_v, rA, rB, sA, sB):
        c = lax.axis_index("c")
        s = lax.axis_index("s")
        wid = s * NC + c

        def step(ch):
            base = ch * CHUNK
            pltpu.sync_copy(src_hbm.at[pl.ds(base, CHUNK)], idx_v)
            dA = pltpu.async_copy(tA.at[idx_v], rA, sA)
            dB = pltpu.async_copy(tB.at[idx_v], rB, sB)
            dA.wait()
            pltpu.sync_copy(rA, oA.at[pl.ds(base, CHUNK)])
            dB.wait()
            pltpu.sync_copy(rB, oB.at[pl.ds(base, CHUNK)])

        _chunk_loop(nch, wid, NW, step)

    return gat_k


def _norm_from_parts(degp_ref, hn):
    deg = jnp.concatenate(
        [degp_ref[0, :hn, 0:1], degp_ref[1, :hn, 0:1]], axis=0)
    return 1.0 / jnp.sqrt(jnp.where(deg > 0.0, deg, 1.0))


def _elu(x):
    return jnp.where(x > 0.0, x, jnp.exp(jnp.where(x > 0.0, 0.0, x)) - 1.0)


def _merge_acc(ap, hn):
    return jnp.concatenate([ap[0, :hn], ap[1, :hn]], axis=0)


def _localize_dst(dst, n_nodes):
    e = dst.shape[0]
    hn = n_nodes // NC

    def body(d_ref, out_ref):
        dd = d_ref[...]
        ii = lax.broadcasted_iota(jnp.int32, (1, e), 1)
        sink = hn + (ii & (NDUMMY - 1))
        for c in range(NC):
            dl = dd - c * hn
            ok = (dl >= 0) & (dl < hn)
            out_ref[c:c + 1, :] = jnp.where(ok, dl, sink)

    return pl.pallas_call(
        body, out_shape=jax.ShapeDtypeStruct((NC, e), jnp.int32),
    )(dst.reshape(1, e))


def _stage_a(x_F, x_S, degp, W_F1, b_F1, W_S1, b_S1):
    n = x_F.shape[0]
    hn = n // NC
    h = W_F1.shape[1]

    def body(xf, xs, dp, wf, bf, ws, bs, out):
        norm = _norm_from_parts(dp, hn)
        hf = (jnp.dot(xf[...], wf[...], preferred_element_type=jnp.float32)
              + bf[...]) * norm
        hs = (jnp.dot(xs[...], ws[...], preferred_element_type=jnp.float32)
              + bs[...]) * norm
        out[...] = jnp.concatenate([hf, hs], axis=1)

    return pl.pallas_call(
        body, out_shape=jax.ShapeDtypeStruct((n, 2 * h), jnp.float32),
    )(x_F, x_S, degp, W_F1, b_F1.reshape(1, -1), W_S1, b_S1.reshape(1, -1))


def _stage_b(acc1p, degp, W_F2, b_F2, W_S2, b_S2):
    n = NC * (degp.shape[1] - NDUMMY)
    hn = n // NC
    h = W_F2.shape[1]

    def body(ap, dp, wf, bf, ws, bs, out):
        norm = _norm_from_parts(dp, hn)
        a1 = _merge_acc(ap, hn) * norm
        hh = jnp.maximum(a1, 0.0)
        tf = (jnp.dot(hh[:, :h], wf[...], preferred_element_type=jnp.float32)
              + bf[...]) * norm
        ts = (jnp.dot(hh[:, h:], ws[...], preferred_element_type=jnp.float32)
              + bs[...]) * norm
        out[...] = jnp.concatenate([tf, ts], axis=1)

    return pl.pallas_call(
        body, out_shape=jax.ShapeDtypeStruct((n, 2 * h), jnp.float32),
    )(acc1p, degp, W_F2, b_F2.reshape(1, -1), W_S2, b_S2.reshape(1, -1))


def _stage_c(acc2p, degp, W_fs1, b_fs1, W_fs2, b_fs2,
             W_sf1, b_sf1, W_sf2, b_sf2):
    n = NC * (degp.shape[1] - NDUMMY)
    hn = n // NC
    h = W_fs1.shape[0]

    def body(ap, dp, wfs1, bfs1, wfs2, bfs2, wsf1, bsf1, wsf2, bsf2,
             tab_z, tab_d):
        norm = _norm_from_parts(dp, hn)
        z = _merge_acc(ap, hn) * norm
        zff = z[:, :h]
        zss = z[:, h:]
        tab_z[...] = z

        def dec(zz, w1, b1, w2, b2):
            t = _elu(jnp.dot(zz, w1[...], preferred_element_type=jnp.float32)
                     + b1[...])
            return jnp.dot(t, w2[...], preferred_element_type=jnp.float32) + b2[...]

        df = dec(zff, wfs1, bfs1, wfs2, bfs2)
        ds_ = dec(zss, wsf1, bsf1, wsf2, bsf2)
        tab_d[...] = jnp.concatenate([df, ds_], axis=1)

    sh = jax.ShapeDtypeStruct((n, 2 * h), jnp.float32)
    return pl.pallas_call(
        body, out_shape=(sh, sh),
    )(acc2p, degp, W_fs1, b_fs1.reshape(1, -1), W_fs2, b_fs2.reshape(1, -1),
      W_sf1, b_sf1.reshape(1, -1), W_sf2, b_sf2.reshape(1, -1))


def kernel(x_F, x_S, edge_index, W_F1, b_F1, W_F2, b_F2, W_S1, b_S1,
           W_S2, b_S2, W_fs1, b_fs1, W_fs2, b_fs2, W_sf1, b_sf1,
           W_sf2, b_sf2):
    n = x_F.shape[0]
    e = edge_index.shape[1]
    h = W_F1.shape[1]
    assert e % CHUNK == 0 and n % NC == 0

    src = edge_index[0]
    dst = edge_index[1]

    iota_n = jnp.arange(n, dtype=jnp.int32)
    zeros128 = jnp.zeros((CHUNK, 2 * h), jnp.float32)
    ones_tab = jnp.ones((n, 2 * h), jnp.float32)

    dstloc = _localize_dst(dst, n)

    prop = _make_prop(n, e, 2 * h)
    degp = prop(ones_tab, src, dstloc, iota_n, zeros128)
    table1 = _stage_a(x_F, x_S, degp, W_F1, b_F1, W_S1, b_S1)
    acc1p = prop(table1, src, dstloc, iota_n, zeros128)
    table2 = _stage_b(acc1p, degp, W_F2, b_F2, W_S2, b_S2)
    acc2p = prop(table2, src, dstloc, iota_n, zeros128)
    tab_z, tab_d = _stage_c(
        acc2p, degp, W_fs1, b_fs1, W_fs2, b_fs2, W_sf1, b_sf1, W_sf2, b_sf2)
    oz, od = _make_gather4(n, e, 2 * h)(tab_z, tab_d, src)
    return (oz[:, :h], oz[:, h:], od[:, :h], od[:, h:])


# speedup vs baseline: 3.8272x; 1.1186x over previous
import functools

import jax
import jax.numpy as jnp
from jax import lax
from jax.experimental import pallas as pl
from jax.experimental.pallas import tpu as pltpu
from jax.experimental.pallas import tpu_sc as plsc

NC = 2
NS = 16
NW = NC * NS
L = 16
CHUNK = 128
NDUMMY = 64


def _mesh():
    return plsc.VectorSubcoreMesh(core_axis_name="c", subcore_axis_name="s")


def _chunk_loop(n_chunks, wid, nworkers, step):
    full_t = n_chunks // nworkers
    rem = n_chunks % nworkers

    def body(t, _):
        step(t * nworkers + wid)
        return 0

    lax.fori_loop(0, full_t, body, 0)
    if rem:
        @pl.when(wid < rem)
        def _():
            step(full_t * nworkers + wid)


def _block_loop(n_rows, s, fn):
    nblk = -(-n_rows // CHUNK)
    lastbase = n_rows - CHUNK

    def run(blk):
        fn(jnp.minimum(blk * CHUNK, lastbase))

    _chunk_loop(nblk, s, NS, run)


def _sp_zero(iota_hbm, zeros_v, ids_v, acc_sh, s, n_rows):
    def z(base):
        pltpu.sync_copy(iota_hbm.at[pl.ds(base, CHUNK)], ids_v)
        pltpu.sync_copy(zeros_v, acc_sh.at[ids_v])

    _block_loop(n_rows, s, z)


def _sp_writeout(iota_hbm, rows_v, ids_v, acc_sh, out_hbm, c, s, n_rows,
                 sem):
    def w(base):
        pltpu.sync_copy(iota_hbm.at[pl.ds(base, CHUNK)], ids_v)
        pltpu.async_copy(acc_sh.at[ids_v], rows_v, sem).wait()
        pltpu.sync_copy(rows_v, out_hbm.at[c, pl.ds(base, CHUNK)])

    _block_loop(n_rows, s, w)


def _sp_stage_in(iota_hbm, rows_v, ids_v, tab_hbm, tab_sh, s, n_rows):
    def w(base):
        pltpu.sync_copy(iota_hbm.at[pl.ds(base, CHUNK)], ids_v)
        pltpu.sync_copy(tab_hbm.at[pl.ds(base, CHUNK)], rows_v)
        pltpu.sync_copy(rows_v, tab_sh.at[ids_v])

    _block_loop(n_rows, s, w)


def _make_deg(n_nodes, n_edges):
    nch = n_edges // CHUNK

    @functools.partial(
        pl.kernel,
        out_type=jax.ShapeDtypeStruct((NC, n_nodes, L), jnp.float32),
        mesh=_mesh(),
        scratch_types=[
            pltpu.VMEM((CHUNK,), jnp.int32),
            pltpu.VMEM((CHUNK,), jnp.int32),
            pltpu.VMEM((CHUNK, L), jnp.float32),
            pltpu.VMEM((CHUNK, L), jnp.float32),
            pltpu.VMEM((CHUNK, L), jnp.float32),
            pltpu.VMEM_SHARED((n_nodes, L), jnp.float32),
            pltpu.SemaphoreType.DMA,
        ],
    )
    def deg_k(dst_hbm, iota_hbm, ones_hbm, zeros_hbm, out_hbm,
              idx_v, ids_v, ones_v, zeros_v, rows_v, acc_sh, sem):
        c = lax.axis_index("c")
        s = lax.axis_index("s")
        wid = s * NC + c

        pltpu.sync_copy(ones_hbm, ones_v)
        pltpu.sync_copy(zeros_hbm, zeros_v)
        _sp_zero(iota_hbm, zeros_v, ids_v, acc_sh, s, n_nodes)
        plsc.subcore_barrier()

        def step(ch):
            base = ch * CHUNK
            pltpu.sync_copy(dst_hbm.at[pl.ds(base, CHUNK)], idx_v)
            pltpu.sync_copy(ones_v, acc_sh.at[idx_v], add=True)

        _chunk_loop(nch, wid, NW, step)

        plsc.subcore_barrier()
        _sp_writeout(iota_hbm, rows_v, ids_v, acc_sh, out_hbm, c, s,
                     n_nodes, sem)

    return deg_k


def _make_prop(n_nodes, n_edges, width):
    nch = n_edges // CHUNK
    hn = n_nodes // NC
    acc_rows = hn + NDUMMY

    @functools.partial(
        pl.kernel,
        out_type=jax.ShapeDtypeStruct((NC, acc_rows, width), jnp.float32),
        mesh=_mesh(),
        scratch_types=[
            pltpu.VMEM((CHUNK,), jnp.int32),
            pltpu.VMEM((CHUNK,), jnp.int32),
            pltpu.VMEM((CHUNK,), jnp.int32),
            pltpu.VMEM((CHUNK, width), jnp.float32),
            pltpu.VMEM((CHUNK, width), jnp.float32),
            pltpu.VMEM_SHARED((acc_rows, width), jnp.float32),
            pltpu.SemaphoreType.DMA,
        ],
    )
    def prop_k(table_hbm, src_hbm, dstloc_hbm, iota_hbm, zeros_hbm,
               out_hbm, sidx_v, didx_v, ids_v, rows_v, zeros_v, acc_sh,
               sem):
        c = lax.axis_index("c")
        s = lax.axis_index("s")

        pltpu.sync_copy(zeros_hbm, zeros_v)
        _sp_zero(iota_hbm, zeros_v, ids_v, acc_sh, s, acc_rows)
        plsc.subcore_barrier()

        def step(ch):
            base = ch * CHUNK
            pltpu.sync_copy(src_hbm.at[pl.ds(base, CHUNK)], sidx_v)
            pltpu.sync_copy(dstloc_hbm.at[c, pl.ds(base, CHUNK)], didx_v)
            pltpu.async_copy(table_hbm.at[sidx_v], rows_v, sem).wait()
            pltpu.sync_copy(rows_v, acc_sh.at[didx_v], add=True)

        _chunk_loop(nch, s, NS, step)

        plsc.subcore_barrier()
        _sp_writeout(iota_hbm, rows_v, ids_v, acc_sh, out_hbm, c, s,
                     acc_rows, sem)

    return prop_k


def _make_deg128(n_nodes, n_edges, width):
    nch = n_edges // CHUNK
    hn = n_nodes // NC
    acc_rows = hn + NDUMMY

    @functools.partial(
        pl.kernel,
        out_type=jax.ShapeDtypeStruct((NC, acc_rows, width), jnp.float32),
        mesh=_mesh(),
        scratch_types=[
            pltpu.VMEM((CHUNK,), jnp.int32),
            pltpu.VMEM((CHUNK,), jnp.int32),
            pltpu.VMEM((CHUNK, width), jnp.float32),
            pltpu.VMEM((CHUNK, width), jnp.float32),
            pltpu.VMEM((CHUNK, width), jnp.float32),
            pltpu.VMEM_SHARED((acc_rows, width), jnp.float32),
            pltpu.SemaphoreType.DMA,
        ],
    )
    def deg_k(dstloc_hbm, iota_hbm, zeros_hbm, ones_hbm, out_hbm,
              didx_v, ids_v, rows_v, zeros_v, ones_v, acc_sh, sem):
        c = lax.axis_index("c")
        s = lax.axis_index("s")

        pltpu.sync_copy(zeros_hbm, zeros_v)
        pltpu.sync_copy(ones_hbm, ones_v)
        _sp_zero(iota_hbm, zeros_v, ids_v, acc_sh, s, acc_rows)
        plsc.subcore_barrier()

        def step(ch):
            base = ch * CHUNK
            pltpu.sync_copy(dstloc_hbm.at[c, pl.ds(base, CHUNK)], didx_v)
            pltpu.sync_copy(ones_v, acc_sh.at[didx_v], add=True)

        _chunk_loop(nch, s, NS, step)

        plsc.subcore_barrier()
        _sp_writeout(iota_hbm, rows_v, ids_v, acc_sh, out_hbm, c, s,
                     acc_rows, sem)

    return deg_k


def _make_gather4(n_nodes, n_edges, width):
    nch = n_edges // CHUNK
    out = jax.ShapeDtypeStruct((n_edges, width), jnp.float32)

    @functools.partial(
        pl.kernel,
        out_type=(out, out),
        mesh=_mesh(),
        scratch_types=[
            pltpu.VMEM((CHUNK,), jnp.int32),
            pltpu.VMEM((CHUNK, width), jnp.float32),
            pltpu.VMEM((CHUNK, width), jnp.float32),
            pltpu.SemaphoreType.DMA,
            pltpu.SemaphoreType.DMA,
        ],
    )
    def gat_k(tA, tB, src_hbm, oA, oB, idx_v, rA, rB, sA, sB):
        c = lax.axis_index("c")
        s = lax.axis_index("s")
        wid = s * NC + c

        def step(ch):
            base = ch * CHUNK
            pltpu.sync_copy(src_hbm.at[pl.ds(base, CHUNK)], idx_v)
            dA = pltpu.async_copy(tA.at[idx_v], rA, sA)
            dB = pltpu.async_copy(tB.at[idx_v], rB, sB)
            dA.wait()
            pltpu.sync_copy(rA, oA.at[pl.ds(base, CHUNK)])
            dB.wait()
            pltpu.sync_copy(rB, oB.at[pl.ds(base, CHUNK)])

        _chunk_loop(nch, wid, NW, step)

    return gat_k


def _norm_from_parts(degp_ref, hn):
    deg = jnp.concatenate(
        [degp_ref[0, :hn, 0:1], degp_ref[1, :hn, 0:1]], axis=0)
    return 1.0 / jnp.sqrt(jnp.where(deg > 0.0, deg, 1.0))


def _elu(x):
    return jnp.where(x > 0.0, x, jnp.exp(jnp.where(x > 0.0, 0.0, x)) - 1.0)


def _merge_acc(ap, hn):
    return jnp.concatenate([ap[0, :hn], ap[1, :hn]], axis=0)


def _localize_dst(dst, n_nodes):
    e = dst.shape[0]
    hn = n_nodes // NC

    def body(d_ref, out_ref):
        dd = d_ref[...]
        ii = lax.broadcasted_iota(jnp.int32, (1, e), 1)
        sink = hn + (ii & (NDUMMY - 1))
        for c in range(NC):
            dl = dd - c * hn
            ok = (dl >= 0) & (dl < hn)
            out_ref[c:c + 1, :] = jnp.where(ok, dl, sink)

    return pl.pallas_call(
        body, out_shape=jax.ShapeDtypeStruct((NC, e), jnp.int32),
    )(dst.reshape(1, e))


def _stage_a(x_F, x_S, degp, W_F1, b_F1, W_S1, b_S1):
    n = x_F.shape[0]
    hn = n // NC
    h = W_F1.shape[1]

    def body(xf, xs, dp, wf, bf, ws, bs, out):
        norm = _norm_from_parts(dp, hn)
        hf = (jnp.dot(xf[...], wf[...], preferred_element_type=jnp.float32)
              + bf[...]) * norm
        hs = (jnp.dot(xs[...], ws[...], preferred_element_type=jnp.float32)
              + bs[...]) * norm
        out[...] = jnp.concatenate([hf, hs], axis=1)

    return pl.pallas_call(
        body, out_shape=jax.ShapeDtypeStruct((n, 2 * h), jnp.float32),
    )(x_F, x_S, degp, W_F1, b_F1.reshape(1, -1), W_S1, b_S1.reshape(1, -1))


def _stage_b(acc1p, degp, W_F2, b_F2, W_S2, b_S2):
    n = NC * (degp.shape[1] - NDUMMY)
    hn = n // NC
    h = W_F2.shape[1]

    def body(ap, dp, wf, bf, ws, bs, out):
        norm = _norm_from_parts(dp, hn)
        a1 = _merge_acc(ap, hn) * norm
        hh = jnp.maximum(a1, 0.0)
        tf = (jnp.dot(hh[:, :h], wf[...], preferred_element_type=jnp.float32)
              + bf[...]) * norm
        ts = (jnp.dot(hh[:, h:], ws[...], preferred_element_type=jnp.float32)
              + bs[...]) * norm
        out[...] = jnp.concatenate([tf, ts], axis=1)

    return pl.pallas_call(
        body, out_shape=jax.ShapeDtypeStruct((n, 2 * h), jnp.float32),
    )(acc1p, degp, W_F2, b_F2.reshape(1, -1), W_S2, b_S2.reshape(1, -1))


def _stage_c(acc2p, degp, W_fs1, b_fs1, W_fs2, b_fs2,
             W_sf1, b_sf1, W_sf2, b_sf2):
    n = NC * (degp.shape[1] - NDUMMY)
    hn = n // NC
    h = W_fs1.shape[0]

    def body(ap, dp, wfs1, bfs1, wfs2, bfs2, wsf1, bsf1, wsf2, bsf2,
             tab_z, tab_d):
        norm = _norm_from_parts(dp, hn)
        z = _merge_acc(ap, hn) * norm
        zff = z[:, :h]
        zss = z[:, h:]
        tab_z[...] = z

        def dec(zz, w1, b1, w2, b2):
            t = _elu(jnp.dot(zz, w1[...], preferred_element_type=jnp.float32)
                     + b1[...])
            return jnp.dot(t, w2[...], preferred_element_type=jnp.float32) + b2[...]

        df = dec(zff, wfs1, bfs1, wfs2, bfs2)
        ds_ = dec(zss, wsf1, bsf1, wsf2, bsf2)
        tab_d[...] = jnp.concatenate([df, ds_], axis=1)

    sh = jax.ShapeDtypeStruct((n, 2 * h), jnp.float32)
    return pl.pallas_call(
        body, out_shape=(sh, sh),
    )(acc2p, degp, W_fs1, b_fs1.reshape(1, -1), W_fs2, b_fs2.reshape(1, -1),
      W_sf1, b_sf1.reshape(1, -1), W_sf2, b_sf2.reshape(1, -1))


def kernel(x_F, x_S, edge_index, W_F1, b_F1, W_F2, b_F2, W_S1, b_S1,
           W_S2, b_S2, W_fs1, b_fs1, W_fs2, b_fs2, W_sf1, b_sf1,
           W_sf2, b_sf2):
    n = x_F.shape[0]
    e = edge_index.shape[1]
    h = W_F1.shape[1]
    assert e % CHUNK == 0 and n % NC == 0

    src = edge_index[0]
    dst = edge_index[1]

    iota_n = jnp.arange(n, dtype=jnp.int32)
    zeros128 = jnp.zeros((CHUNK, 2 * h), jnp.float32)
    ones128 = jnp.ones((CHUNK, 2 * h), jnp.float32)

    dstloc = _localize_dst(dst, n)

    prop = _make_prop(n, e, 2 * h)
    degp = _make_deg128(n, e, 2 * h)(dstloc, iota_n, zeros128, ones128)
    table1 = _stage_a(x_F, x_S, degp, W_F1, b_F1, W_S1, b_S1)
    acc1p = prop(table1, src, dstloc, iota_n, zeros128)
    table2 = _stage_b(acc1p, degp, W_F2, b_F2, W_S2, b_S2)
    acc2p = prop(table2, src, dstloc, iota_n, zeros128)
    tab_z, tab_d = _stage_c(
        acc2p, degp, W_fs1, b_fs1, W_fs2, b_fs2, W_sf1, b_sf1, W_sf2, b_sf2)
    oz, od = _make_gather4(n, e, 2 * h)(tab_z, tab_d, src)
    return (oz[:, :h], oz[:, h:], od[:, :h], od[:, h:])
